# Initial kernel scaffold; baseline (speedup 1.0000x reference)
import jax
import jax.numpy as jnp
from jax.experimental import pallas as pl


def kernel(x, edge_index, batch, params):
    raise NotImplementedError("write your pallas kernel here")


# speedup vs baseline: 2.6170x; 2.6170x over previous
import functools

import jax
import jax.numpy as jnp
from jax import lax
from jax.experimental import pallas as pl
from jax.experimental.pallas import tpu as pltpu
from jax.experimental.pallas import tpu_sc as plsc

NN = 10000
NE = 320000
D = 128
G = 16
EPS = 1e-5

NC, NS = 2, 16
NW = NC * NS
K = 80
NCHUNK = 128
E_W = K * NCHUNK
NE_PAD = NW * E_W
NN_PAD = 10240
R_TILE = NN_PAD // NS


def _edge_agg(h, src_r, dst_r):
    mesh = plsc.VectorSubcoreMesh(core_axis_name="c", subcore_axis_name="s")

    @functools.partial(
        pl.kernel,
        mesh=mesh,
        out_type=jax.ShapeDtypeStruct((NC, NN_PAD, D), jnp.float32),
        scratch_types=[
            pltpu.VMEM((NCHUNK, K), jnp.int32),
            pltpu.VMEM((NCHUNK, K), jnp.int32),
            pltpu.VMEM((K, D), jnp.float32),
            pltpu.VMEM_SHARED((NN_PAD, D), jnp.float32),
            pltpu.SemaphoreType.DMA,
        ],
    )
    def agg_kernel(h_hbm, src_hbm, dst_hbm, out_hbm,
                   src_v, dst_v, rows_v, agg_sh, sem):
        c = lax.axis_index("c")
        s = lax.axis_index("s")
        wid = s * NC + c

        def zb(i, carry):
            for j in range(D // 16):
                rows_v[i, pl.ds(j * 16, 16)] = jnp.zeros((16,), jnp.float32)
            return carry
        lax.fori_loop(0, K, zb, 0)
        for t in range(R_TILE // K):
            pltpu.sync_copy(rows_v, agg_sh.at[pl.ds(s * R_TILE + t * K, K)])
        plsc.subcore_barrier()

        pltpu.sync_copy(src_hbm.at[wid], src_v)
        pltpu.sync_copy(dst_hbm.at[wid], dst_v)

        def body(j, carry):
            pltpu.async_copy(h_hbm.at[src_v.at[j]], rows_v, sem).wait()
            pltpu.sync_copy(rows_v, agg_sh.at[dst_v.at[j]], add=True)
            return carry
        lax.fori_loop(0, NCHUNK, body, 0)

        plsc.subcore_barrier()
        pltpu.sync_copy(agg_sh.at[pl.ds(s * R_TILE, R_TILE)],
                        out_hbm.at[c, pl.ds(s * R_TILE, R_TILE)])

    return agg_kernel(h, src_r, dst_r)[:, :NN]


def _mm(z, W):
    return jnp.dot(z.astype(jnp.bfloat16), W.astype(jnp.bfloat16),
                   preferred_element_type=jnp.float32)


def _bn_cols(z, g, b):
    m = jnp.mean(z, axis=0, keepdims=True)
    v = jnp.mean((z - m) * (z - m), axis=0, keepdims=True)
    return g * (z - m) * lax.rsqrt(v + EPS) + b


def _gin_mlp(z, W1, b1, g1, be1, W2, b2):
    z = _mm(z, W1) + b1
    z = jnp.maximum(_bn_cols(z, g1, be1), 0.0)
    return _mm(z, W2) + b2


def _layer1_body(x_ref, a0, a1, W1, b1, g1, be1, W2, b2, gbn, bbn, vn,
                 out_ref):
    z = x_ref[...] + a0[...] + a1[...]
    z = _gin_mlp(z, W1[...], b1[...], g1[...], be1[...], W2[...], b2[...])
    z = jnp.maximum(_bn_cols(z, gbn[...], bbn[...]), 0.0)
    out_ref[...] = z + vn[...]


def _layer2_body(h, a0, a1, W1, b1, g1, be1, W2, b2, gbn, bbn,
                 bcol, brow, vn, Wv1, bv1, gv1, bev1, Wv2, bv2, gv2, bev2,
                 out_ref):
    z = h[...] + a0[...] + a1[...]
    z = _gin_mlp(z, W1[...], b1[...], g1[...], be1[...], W2[...], b2[...])
    post = jnp.maximum(_bn_cols(z, gbn[...], bbn[...]), 0.0)
    oh = (bcol[...] == lax.broadcasted_iota(jnp.int32, (1, G), 1)
          ).astype(jnp.float32)
    ohT = (lax.broadcasted_iota(jnp.int32, (G, 1), 0) == brow[...]
           ).astype(jnp.float32)
    pooled = jnp.dot(ohT, post, preferred_element_type=jnp.float32, precision=lax.Precision.HIGHEST)
    v = pooled + vn[...]
    v = _mm(v, Wv1[...]) + bv1[...]
    v = jnp.maximum(_bn_cols(v, gv1[...], bev1[...]), 0.0)
    v = _mm(v, Wv2[...]) + bv2[...]
    v = jnp.maximum(_bn_cols(v, gv2[...], bev2[...]), 0.0)
    out_ref[...] = post + jnp.dot(oh, v, preferred_element_type=jnp.float32, precision=lax.Precision.HIGHEST)


def _layer3_body(h, a0, a1, W1, b1, g1, be1, W2, b2, gbn, bbn, brow,
                 out_ref):
    z = h[...] + a0[...] + a1[...]
    z = _gin_mlp(z, W1[...], b1[...], g1[...], be1[...], W2[...], b2[...])
    post = _bn_cols(z, gbn[...], bbn[...])
    ohT = (lax.broadcasted_iota(jnp.int32, (G, 1), 0) == brow[...]
           ).astype(jnp.float32)
    sums = jnp.dot(ohT, post, preferred_element_type=jnp.float32, precision=lax.Precision.HIGHEST)
    counts = jnp.sum(ohT, axis=1, keepdims=True)
    out_ref[...] = sums / jnp.maximum(counts, 1.0)


def _row(a):
    return a.reshape(1, -1)


def kernel(x, edge_index, batch, params):
    pad = NE_PAD - NE
    src_r = jnp.concatenate(
        [edge_index[0], jnp.zeros((pad,), jnp.int32)]).reshape(NW, NCHUNK, K)
    dst_r = jnp.concatenate(
        [edge_index[1], jnp.full((pad,), NN, jnp.int32)]).reshape(NW, NCHUNK, K)
    bcol = batch.reshape(NN, 1)
    brow = batch.reshape(1, NN)

    p1 = params["conv1"]
    bn1 = params["bn1"]
    vn = params["vn_emb"]
    vm = params["vmlp"]

    agg = _edge_agg(x, src_r, dst_r)
    h2 = pl.pallas_call(
        _layer1_body,
        out_shape=jax.ShapeDtypeStruct((NN, D), jnp.float32),
    )(x, agg[0], agg[1],
      p1["W1"], _row(p1["b1"]), _row(p1["g1"]), _row(p1["be1"]),
      p1["W2"], _row(p1["b2"]), _row(bn1["g"]), _row(bn1["b"]), vn)

    p2 = params["convs"][0]
    bn2 = params["bns"][0]
    agg = _edge_agg(h2, src_r, dst_r)
    h3 = pl.pallas_call(
        _layer2_body,
        out_shape=jax.ShapeDtypeStruct((NN, D), jnp.float32),
    )(h2, agg[0], agg[1],
      p2["W1"], _row(p2["b1"]), _row(p2["g1"]), _row(p2["be1"]),
      p2["W2"], _row(p2["b2"]), _row(bn2["g"]), _row(bn2["b"]),
      bcol, brow, vn,
      vm["W1"], _row(vm["b1"]), _row(vm["g1"]), _row(vm["be1"]),
      vm["W2"], _row(vm["b2"]), _row(vm["g2"]), _row(vm["be2"]))

    p3 = params["convs"][1]
    bn3 = params["bns"][1]
    agg = _edge_agg(h3, src_r, dst_r)
    out = pl.pallas_call(
        _layer3_body,
        out_shape=jax.ShapeDtypeStruct((G, D), jnp.float32),
    )(h3, agg[0], agg[1],
      p3["W1"], _row(p3["b1"]), _row(p3["g1"]), _row(p3["be1"]),
      p3["W2"], _row(p3["b2"]), _row(bn3["g"]), _row(bn3["b"]), brow)
    return out
